# baseline (speedup 1.0000x reference)
import functools

import jax
import jax.numpy as jnp
import numpy as np
from jax import lax
from jax.experimental import pallas as pl
from jax.experimental.pallas import tpu as pltpu
from jax.experimental.pallas import tpu_sc as plsc

B, S, L, F = 1024, 8, 200, 3
EDIM = 16
EXT = F * EDIM
G, GL = 4, 3
GM = G * GL
MC = 2 ** GL
GC = G * MC
Q = 8

N_LT = B * L * F
N_IT = B * S * F
N_ROWS = N_LT + N_IT

NW = 32
ROWS_PER_W = N_ROWS // NW
SUB = 128
NSUB = 12
CHUNK = SUB * NSUB
NCHUNK = ROWS_PER_W // CHUNK
PROW = CHUNK * EDIM // 128
LT_CHUNKS = N_LT // CHUNK
P_LT = N_LT * EDIM // 128
P_IT = N_IT * EDIM // 128


def _sc_gather_packed(table, ids):
    mesh = plsc.VectorSubcoreMesh(core_axis_name="c", subcore_axis_name="s")

    @functools.partial(
        pl.kernel,
        out_type=[jax.ShapeDtypeStruct((P_LT, 128), jnp.float32),
                  jax.ShapeDtypeStruct((P_IT, 128), jnp.float32)],
        mesh=mesh,
        scratch_types=[
            pltpu.VMEM((CHUNK,), jnp.int32),
            pltpu.VMEM((CHUNK, EDIM), jnp.float32),
            pltpu.VMEM((PROW, 128), jnp.float32),
            pltpu.SemaphoreType.DMA,
        ],
        compiler_params=pltpu.CompilerParams(use_tc_tiling_on_sc=False),
    )
    def k(table_hbm, ids_hbm, lt_hbm, it_hbm, idx_v, rows_v, packed_v, sem):
        wid = lax.axis_index("s") * 2 + lax.axis_index("c")

        def body(i, carry):
            c = wid * NCHUNK + i
            pltpu.sync_copy(ids_hbm.at[pl.ds(c * CHUNK, CHUNK)], idx_v)
            copies = []
            for j in range(NSUB):
                copies.append(
                    pltpu.async_copy(
                        table_hbm.at[idx_v.at[pl.ds(j * SUB, SUB)]],
                        rows_v.at[pl.ds(j * SUB, SUB)], sem))
            for cp in copies:
                cp.wait()

            def repack(r, c2):
                for u in range(8):
                    packed_v[r, pl.ds(16 * u, 16)] = rows_v[8 * r + u, :]
                return c2

            lax.fori_loop(0, PROW, repack, 0)

            @pl.when(c < LT_CHUNKS)
            def _():
                pltpu.sync_copy(packed_v, lt_hbm.at[pl.ds(c * PROW, PROW)])

            @pl.when(c >= LT_CHUNKS)
            def _():
                pltpu.sync_copy(
                    packed_v,
                    it_hbm.at[pl.ds((c - LT_CHUNKS) * PROW, PROW)])

            return carry

        lax.fori_loop(0, NCHUNK, body, 0)

    return k(table, ids)


BB = 64
NR = BB // Q
B8 = B // Q


def _tc_combine(xlt, xit, maskt, w_all, tg, e8):
    def kern(x_ref, ix_ref, mk_ref, w_ref, tg_ref, e8_ref, out_ref):
        x3 = x_ref[...]
        ix3 = ix_ref[...]
        mkt = mk_ref[0]
        w = w_ref[...]
        tg = tg_ref[...]
        e8 = e8_ref[...]
        for r in range(NR):
            xr = x3[:, r, :]
            ixr = ix3[:, r, :]
            y = jnp.zeros((L, Q * GM), jnp.float32)
            yi = jnp.zeros((S, Q * GM), jnp.float32)
            for f in range(F):
                y = y + lax.dot_general(
                    xr[L * f:L * (f + 1), :], w[f],
                    (((1,), (0,)), ((), ())),
                    preferred_element_type=jnp.float32)
                yi = yi + lax.dot_general(
                    ixr[S * f:S * (f + 1), :], w[f],
                    (((1,), (0,)), ((), ())),
                    preferred_element_type=jnp.float32)
            sb = jnp.where(y > 0, 1.0, -1.0)
            oh = (lax.dot_general(sb, tg, (((1,), (0,)), ((), ())),
                                  preferred_element_type=jnp.float32)
                  > GL - 0.5).astype(jnp.float32)
            sbi = jnp.where(yi > 0, 1.0, -1.0)
            ohi = (lax.dot_general(sbi, tg, (((1,), (0,)), ((), ())),
                                   preferred_element_type=jnp.float32)
                   > GL - 0.5).astype(jnp.float32)
            valid = jnp.sum(oh, axis=0)
            inv = (1.0 / G) / jnp.maximum(valid, 1.0)
            mk_big = lax.dot_general(
                mkt[:, Q * r:Q * (r + 1)], e8, (((1,), (0,)), ((), ())),
                preferred_element_type=jnp.float32)
            ohm = oh * mk_big
            its = ohi * inv[None, :]
            oqs = []
            for q in range(Q):
                ohm_q = ohm[:, GC * q:GC * (q + 1)]
                it_q = its[:, GC * q:GC * (q + 1)]
                ofs = []
                for f in range(F):
                    x_fq = xr[L * f:L * (f + 1),
                              EDIM * q:EDIM * (q + 1)]
                    a_fq = lax.dot_general(
                        ohm_q, x_fq, (((0,), (0,)), ((), ())),
                        preferred_element_type=jnp.float32)
                    ofs.append(lax.dot_general(
                        it_q, a_fq, (((1,), (0,)), ((), ())),
                        preferred_element_type=jnp.float32))
                oqs.append(jnp.concatenate(ofs, axis=1))
            out_ref[r] = jnp.stack(oqs, axis=0)

    return pl.pallas_call(
        kern,
        grid=(B // BB,),
        in_specs=[
            pl.BlockSpec((F * L, NR, 128), lambda i: (0, i, 0)),
            pl.BlockSpec((F * S, NR, 128), lambda i: (0, i, 0)),
            pl.BlockSpec((1, L, BB), lambda i: (i, 0, 0)),
            pl.BlockSpec((F, 128, Q * GM), lambda i: (0, 0, 0)),
            pl.BlockSpec((Q * GM, Q * GC), lambda i: (0, 0)),
            pl.BlockSpec((Q, Q * GC), lambda i: (0, 0)),
        ],
        out_specs=pl.BlockSpec((NR, Q, S, EXT), lambda i: (i, 0, 0, 0)),
        out_shape=jax.ShapeDtypeStruct((B8, Q, S, EXT), jnp.float32),
    )(xlt, xit, maskt, w_all, tg, e8)


def _make_t_sel():
    t = np.zeros((GM, GC), np.float32)
    for g in range(G):
        for c in range(MC):
            for m in range(GL):
                t[GL * g + m, MC * g + c] = 1.0 if (c >> m) & 1 else -1.0
    return t


_TG = np.kron(np.eye(Q, dtype=np.float32), _make_t_sel())
_E8 = np.kron(np.eye(Q, dtype=np.float32),
              np.ones((1, GC), np.float32))


def kernel(item_ids, longterm_ids, longterm_mask, embed_table, H):
    lt_t = jnp.transpose(longterm_ids, (2, 1, 0))
    it_t = jnp.transpose(item_ids, (2, 1, 0))
    ids = jnp.concatenate([lt_t.reshape(-1), it_t.reshape(-1)])
    out_lt, out_it = _sc_gather_packed(embed_table, ids.astype(jnp.int32))
    xlt = out_lt.reshape(F * L, B8, 128)
    xit = out_it.reshape(F * S, B8, 128)
    maskt = longterm_mask.T.astype(jnp.float32)
    maskt = maskt.reshape(L, B // BB, BB).transpose(1, 0, 2)
    h2 = H.reshape(EXT, GM)
    w_all = jnp.stack([
        jnp.kron(jnp.eye(Q, dtype=jnp.float32), h2[EDIM * f:EDIM * (f + 1)])
        for f in range(F)])
    out = _tc_combine(xlt, xit, maskt, w_all, jnp.asarray(_TG),
                      jnp.asarray(_E8))
    return out.reshape(B, S, EXT)


# speedup vs baseline: 1.7719x; 1.7719x over previous
import functools

import jax
import jax.numpy as jnp
import numpy as np
from jax import lax
from jax.experimental import pallas as pl
from jax.experimental.pallas import tpu as pltpu
from jax.experimental.pallas import tpu_sc as plsc

B, S, L, F = 1024, 8, 200, 3
EDIM = 16
EXT = F * EDIM
G, GL = 4, 3
GM = G * GL
MC = 2 ** GL
GC = G * MC
Q = 8

N_LT = B * L * F
N_IT = B * S * F
N_ROWS = N_LT + N_IT

NW = 32
ROWS_PER_W = N_ROWS // NW
SUB = 128
NSUB = 12
CHUNK = SUB * NSUB
NCHUNK = ROWS_PER_W // CHUNK
PROW = CHUNK * EDIM // 128
LT_CHUNKS = N_LT // CHUNK
P_LT = N_LT * EDIM // 128
P_IT = N_IT * EDIM // 128


def _sc_gather_packed(table, ids):
    mesh = plsc.VectorSubcoreMesh(core_axis_name="c", subcore_axis_name="s")

    @functools.partial(
        pl.kernel,
        out_type=[jax.ShapeDtypeStruct((P_LT, 128), jnp.float32),
                  jax.ShapeDtypeStruct((P_IT, 128), jnp.float32)],
        mesh=mesh,
        scratch_types=[
            pltpu.VMEM((CHUNK,), jnp.int32),
            pltpu.VMEM((CHUNK, EDIM), jnp.float32),
            pltpu.VMEM((PROW, 128), jnp.float32),
            pltpu.SemaphoreType.DMA,
        ],
        compiler_params=pltpu.CompilerParams(use_tc_tiling_on_sc=False),
    )
    def k(table_hbm, ids_hbm, lt_hbm, it_hbm, idx_v, rows_v, packed_v, sem):
        wid = lax.axis_index("s") * 2 + lax.axis_index("c")

        def body(i, carry):
            c = wid * NCHUNK + i
            pltpu.sync_copy(ids_hbm.at[pl.ds(c * CHUNK, CHUNK)], idx_v)
            copies = []
            for j in range(NSUB):
                copies.append(
                    pltpu.async_copy(
                        table_hbm.at[idx_v.at[pl.ds(j * SUB, SUB)]],
                        rows_v.at[pl.ds(j * SUB, SUB)], sem))
            for cp in copies:
                cp.wait()

            def repack(r, c2):
                for u in range(8):
                    packed_v[r, pl.ds(16 * u, 16)] = rows_v[8 * r + u, :]
                return c2

            lax.fori_loop(0, PROW, repack, 0)

            @pl.when(c < LT_CHUNKS)
            def _():
                pltpu.sync_copy(packed_v, lt_hbm.at[pl.ds(c * PROW, PROW)])

            @pl.when(c >= LT_CHUNKS)
            def _():
                pltpu.sync_copy(
                    packed_v,
                    it_hbm.at[pl.ds((c - LT_CHUNKS) * PROW, PROW)])

            return carry

        lax.fori_loop(0, NCHUNK, body, 0)

    return k(table, ids)


BB = 64
NR = BB // Q
B8 = B // Q


def _tc_combine(xlt, xit, maskt, w_all, tg, e8, bd):
    def kern(x_ref, ix_ref, mk_ref, w_ref, tg_ref, e8_ref, bd_ref, out_ref):
        x3 = x_ref[...]
        ix3 = ix_ref[...]
        mkt = mk_ref[0]
        w = w_ref[...]
        tg = tg_ref[...]
        e8 = e8_ref[...]
        for r in range(NR):
            xr = x3[:, r, :]
            ixr = ix3[:, r, :]
            y = jnp.zeros((L, Q * GM), jnp.float32)
            yi = jnp.zeros((S, Q * GM), jnp.float32)
            for f in range(F):
                y = y + lax.dot_general(
                    xr[L * f:L * (f + 1), :], w[f],
                    (((1,), (0,)), ((), ())),
                    preferred_element_type=jnp.float32)
                yi = yi + lax.dot_general(
                    ixr[S * f:S * (f + 1), :], w[f],
                    (((1,), (0,)), ((), ())),
                    preferred_element_type=jnp.float32)
            sb = jnp.where(y > 0, 1.0, -1.0)
            oh = (lax.dot_general(sb, tg, (((1,), (0,)), ((), ())),
                                  preferred_element_type=jnp.float32)
                  > GL - 0.5).astype(jnp.float32)
            sbi = jnp.where(yi > 0, 1.0, -1.0)
            ohi = (lax.dot_general(sbi, tg, (((1,), (0,)), ((), ())),
                                   preferred_element_type=jnp.float32)
                   > GL - 0.5).astype(jnp.float32)
            valid = jnp.sum(oh, axis=0)
            inv = (1.0 / G) / jnp.maximum(valid, 1.0)
            mk_big = lax.dot_general(
                mkt[:, Q * r:Q * (r + 1)], e8, (((1,), (0,)), ((), ())),
                preferred_element_type=jnp.float32)
            ohm = oh * mk_big
            its = ohi * inv[None, :]
            bd = bd_ref[...]
            ofs = []
            for f in range(F):
                xf = xr[L * f:L * (f + 1), :]
                a_full = lax.dot_general(
                    ohm, xf, (((0,), (0,)), ((), ())),
                    preferred_element_type=jnp.float32)
                ofs.append(lax.dot_general(
                    its, a_full * bd, (((1,), (0,)), ((), ())),
                    preferred_element_type=jnp.float32))
            out_ref[r] = jnp.stack(ofs, axis=1)

    return pl.pallas_call(
        kern,
        grid=(B // BB,),
        in_specs=[
            pl.BlockSpec((F * L, NR, 128), lambda i: (0, i, 0)),
            pl.BlockSpec((F * S, NR, 128), lambda i: (0, i, 0)),
            pl.BlockSpec((1, L, BB), lambda i: (i, 0, 0)),
            pl.BlockSpec((F, 128, Q * GM), lambda i: (0, 0, 0)),
            pl.BlockSpec((Q * GM, Q * GC), lambda i: (0, 0)),
            pl.BlockSpec((Q, Q * GC), lambda i: (0, 0)),
            pl.BlockSpec((Q * GC, 128), lambda i: (0, 0)),
        ],
        out_specs=pl.BlockSpec((NR, S, F, 128), lambda i: (i, 0, 0, 0)),
        out_shape=jax.ShapeDtypeStruct((B8, S, F, 128), jnp.float32),
    )(xlt, xit, maskt, w_all, tg, e8, bd)


def _make_t_sel():
    t = np.zeros((GM, GC), np.float32)
    for g in range(G):
        for c in range(MC):
            for m in range(GL):
                t[GL * g + m, MC * g + c] = 1.0 if (c >> m) & 1 else -1.0
    return t


_TG = np.kron(np.eye(Q, dtype=np.float32), _make_t_sel())
_E8 = np.kron(np.eye(Q, dtype=np.float32),
              np.ones((1, GC), np.float32))
_BD = np.kron(np.eye(Q, dtype=np.float32),
              np.ones((GC, EDIM), np.float32))


def kernel(item_ids, longterm_ids, longterm_mask, embed_table, H):
    lt_t = jnp.transpose(longterm_ids, (2, 1, 0))
    it_t = jnp.transpose(item_ids, (2, 1, 0))
    ids = jnp.concatenate([lt_t.reshape(-1), it_t.reshape(-1)])
    out_lt, out_it = _sc_gather_packed(embed_table, ids.astype(jnp.int32))
    xlt = out_lt.reshape(F * L, B8, 128)
    xit = out_it.reshape(F * S, B8, 128)
    maskt = longterm_mask.T.astype(jnp.float32)
    maskt = maskt.reshape(L, B // BB, BB).transpose(1, 0, 2)
    h2 = H.reshape(EXT, GM)
    w_all = jnp.stack([
        jnp.kron(jnp.eye(Q, dtype=jnp.float32), h2[EDIM * f:EDIM * (f + 1)])
        for f in range(F)])
    out = _tc_combine(xlt, xit, maskt, w_all, jnp.asarray(_TG),
                      jnp.asarray(_E8), jnp.asarray(_BD))
    out5 = out.reshape(B8, S, F, Q, EDIM).transpose(0, 3, 1, 2, 4)
    return out5.reshape(B, S, EXT)


# speedup vs baseline: 1.7792x; 1.0042x over previous
import functools

import jax
import jax.numpy as jnp
import numpy as np
from jax import lax
from jax.experimental import pallas as pl
from jax.experimental.pallas import tpu as pltpu
from jax.experimental.pallas import tpu_sc as plsc

B, S, L, F = 1024, 8, 200, 3
EDIM = 16
EXT = F * EDIM
G, GL = 4, 3
GM = G * GL
MC = 2 ** GL
GC = G * MC
Q = 8

N_LT = B * L * F
N_IT = B * S * F
N_ROWS = N_LT + N_IT

NW = 32
ROWS_PER_W = N_ROWS // NW
SUB = 128
NSUB = 12
CHUNK = SUB * NSUB
NCHUNK = ROWS_PER_W // CHUNK
PROW = CHUNK * EDIM // 128
LT_CHUNKS = N_LT // CHUNK
P_LT = N_LT * EDIM // 128
P_IT = N_IT * EDIM // 128


def _sc_gather_packed(table, ids):
    mesh = plsc.VectorSubcoreMesh(core_axis_name="c", subcore_axis_name="s")

    @functools.partial(
        pl.kernel,
        out_type=[jax.ShapeDtypeStruct((P_LT, 128), jnp.float32),
                  jax.ShapeDtypeStruct((P_IT, 128), jnp.float32)],
        mesh=mesh,
        scratch_types=[
            pltpu.VMEM((CHUNK,), jnp.int32),
            pltpu.VMEM((CHUNK, EDIM), jnp.float32),
            pltpu.VMEM((PROW, 128), jnp.float32),
            pltpu.SemaphoreType.DMA,
        ],
        compiler_params=pltpu.CompilerParams(use_tc_tiling_on_sc=False),
    )
    def k(table_hbm, ids_hbm, lt_hbm, it_hbm, idx_v, rows_v, packed_v, sem):
        wid = lax.axis_index("s") * 2 + lax.axis_index("c")

        def body(i, carry):
            c = wid * NCHUNK + i
            pltpu.sync_copy(ids_hbm.at[pl.ds(c * CHUNK, CHUNK)], idx_v)
            copies = []
            for j in range(NSUB):
                copies.append(
                    pltpu.async_copy(
                        table_hbm.at[idx_v.at[pl.ds(j * SUB, SUB)]],
                        rows_v.at[pl.ds(j * SUB, SUB)], sem))
            for cp in copies:
                cp.wait()

            def repack(r, c2):
                for u in range(8):
                    packed_v[r, pl.ds(16 * u, 16)] = rows_v[8 * r + u, :]
                return c2

            lax.fori_loop(0, PROW, repack, 0)

            @pl.when(c < LT_CHUNKS)
            def _():
                pltpu.sync_copy(packed_v, lt_hbm.at[pl.ds(c * PROW, PROW)])

            @pl.when(c >= LT_CHUNKS)
            def _():
                pltpu.sync_copy(
                    packed_v,
                    it_hbm.at[pl.ds((c - LT_CHUNKS) * PROW, PROW)])

            return carry

        lax.fori_loop(0, NCHUNK, body, 0)

    return k(table, ids)


BB = 128
NR = BB // Q
B8 = B // Q


def _tc_combine(xlt, xit, maskt, w_all, tg, e8, bd):
    def kern(x_ref, ix_ref, mk_ref, w_ref, tg_ref, e8_ref, bd_ref, out_ref):
        x3 = x_ref[...]
        ix3 = ix_ref[...]
        mkt = mk_ref[0]
        w = w_ref[...]
        tg = tg_ref[...]
        e8 = e8_ref[...]
        for r in range(NR):
            xr = x3[:, r, :]
            ixr = ix3[:, r, :]
            y = jnp.zeros((L, Q * GM), jnp.float32)
            yi = jnp.zeros((S, Q * GM), jnp.float32)
            for f in range(F):
                y = y + lax.dot_general(
                    xr[L * f:L * (f + 1), :], w[f],
                    (((1,), (0,)), ((), ())),
                    preferred_element_type=jnp.float32)
                yi = yi + lax.dot_general(
                    ixr[S * f:S * (f + 1), :], w[f],
                    (((1,), (0,)), ((), ())),
                    preferred_element_type=jnp.float32)
            sb = jnp.where(y > 0, 1.0, -1.0)
            oh = (lax.dot_general(sb, tg, (((1,), (0,)), ((), ())),
                                  preferred_element_type=jnp.float32)
                  > GL - 0.5).astype(jnp.float32)
            sbi = jnp.where(yi > 0, 1.0, -1.0)
            ohi = (lax.dot_general(sbi, tg, (((1,), (0,)), ((), ())),
                                   preferred_element_type=jnp.float32)
                   > GL - 0.5).astype(jnp.float32)
            valid = jnp.sum(oh, axis=0)
            inv = (1.0 / G) / jnp.maximum(valid, 1.0)
            mk_big = lax.dot_general(
                mkt[:, Q * r:Q * (r + 1)], e8, (((1,), (0,)), ((), ())),
                preferred_element_type=jnp.float32)
            ohm = oh * mk_big
            its = ohi * inv[None, :]
            bd = bd_ref[...]
            ofs = []
            for f in range(F):
                xf = xr[L * f:L * (f + 1), :]
                a_full = lax.dot_general(
                    ohm, xf, (((0,), (0,)), ((), ())),
                    preferred_element_type=jnp.float32)
                ofs.append(lax.dot_general(
                    its, a_full * bd, (((1,), (0,)), ((), ())),
                    preferred_element_type=jnp.float32))
            out_ref[r] = jnp.stack(ofs, axis=1)

    return pl.pallas_call(
        kern,
        grid=(B // BB,),
        in_specs=[
            pl.BlockSpec((F * L, NR, 128), lambda i: (0, i, 0)),
            pl.BlockSpec((F * S, NR, 128), lambda i: (0, i, 0)),
            pl.BlockSpec((1, L, BB), lambda i: (i, 0, 0)),
            pl.BlockSpec((F, 128, Q * GM), lambda i: (0, 0, 0)),
            pl.BlockSpec((Q * GM, Q * GC), lambda i: (0, 0)),
            pl.BlockSpec((Q, Q * GC), lambda i: (0, 0)),
            pl.BlockSpec((Q * GC, 128), lambda i: (0, 0)),
        ],
        out_specs=pl.BlockSpec((NR, S, F, 128), lambda i: (i, 0, 0, 0)),
        out_shape=jax.ShapeDtypeStruct((B8, S, F, 128), jnp.float32),
    )(xlt, xit, maskt, w_all, tg, e8, bd)


def _make_t_sel():
    t = np.zeros((GM, GC), np.float32)
    for g in range(G):
        for c in range(MC):
            for m in range(GL):
                t[GL * g + m, MC * g + c] = 1.0 if (c >> m) & 1 else -1.0
    return t


_TG = np.kron(np.eye(Q, dtype=np.float32), _make_t_sel())
_E8 = np.kron(np.eye(Q, dtype=np.float32),
              np.ones((1, GC), np.float32))
_BD = np.kron(np.eye(Q, dtype=np.float32),
              np.ones((GC, EDIM), np.float32))


def kernel(item_ids, longterm_ids, longterm_mask, embed_table, H):
    lt_t = jnp.transpose(longterm_ids, (2, 1, 0))
    it_t = jnp.transpose(item_ids, (2, 1, 0))
    ids = jnp.concatenate([lt_t.reshape(-1), it_t.reshape(-1)])
    out_lt, out_it = _sc_gather_packed(embed_table, ids.astype(jnp.int32))
    xlt = out_lt.reshape(F * L, B8, 128)
    xit = out_it.reshape(F * S, B8, 128)
    maskt = longterm_mask.T.astype(jnp.float32)
    maskt = maskt.reshape(L, B // BB, BB).transpose(1, 0, 2)
    h2 = H.reshape(EXT, GM)
    w_all = jnp.stack([
        jnp.kron(jnp.eye(Q, dtype=jnp.float32), h2[EDIM * f:EDIM * (f + 1)])
        for f in range(F)])
    out = _tc_combine(xlt, xit, maskt, w_all, jnp.asarray(_TG),
                      jnp.asarray(_E8), jnp.asarray(_BD))
    out5 = out.reshape(B8, S, F, Q, EDIM).transpose(0, 3, 1, 2, 4)
    return out5.reshape(B, S, EXT)


# speedup vs baseline: 1.9350x; 1.0876x over previous
import functools

import jax
import jax.numpy as jnp
import numpy as np
from jax import lax
from jax.experimental import pallas as pl
from jax.experimental.pallas import tpu as pltpu
from jax.experimental.pallas import tpu_sc as plsc

B, S, L, F = 1024, 8, 200, 3
EDIM = 16
EXT = F * EDIM
G, GL = 4, 3
GM = G * GL
MC = 2 ** GL
GC = G * MC
Q = 8

N_LT = B * L * F
N_IT = B * S * F
N_ROWS = N_LT + N_IT

NW = 32
ROWS_PER_W = N_ROWS // NW
SUB = 128
NSUB = 12
CHUNK = SUB * NSUB
NCHUNK = ROWS_PER_W // CHUNK
PROW = CHUNK * EDIM // 128
LT_CHUNKS = N_LT // CHUNK
P_LT = N_LT * EDIM // 128
P_IT = N_IT * EDIM // 128


def _sc_gather_packed(table, ids):
    mesh = plsc.VectorSubcoreMesh(core_axis_name="c", subcore_axis_name="s")

    @functools.partial(
        pl.kernel,
        out_type=[jax.ShapeDtypeStruct((P_LT, 128), jnp.float32),
                  jax.ShapeDtypeStruct((P_IT, 128), jnp.float32)],
        mesh=mesh,
        scratch_types=[
            pltpu.VMEM((CHUNK,), jnp.int32),
            pltpu.VMEM((CHUNK, EDIM), jnp.float32),
            pltpu.VMEM((PROW, 128), jnp.float32),
            pltpu.SemaphoreType.DMA,
        ],
        compiler_params=pltpu.CompilerParams(use_tc_tiling_on_sc=False),
    )
    def k(table_hbm, ids_hbm, lt_hbm, it_hbm, idx_v, rows_v, packed_v, sem):
        wid = lax.axis_index("s") * 2 + lax.axis_index("c")

        def body(i, carry):
            c = wid * NCHUNK + i
            pltpu.sync_copy(ids_hbm.at[pl.ds(c * CHUNK, CHUNK)], idx_v)
            copies = []
            for j in range(NSUB):
                copies.append(
                    pltpu.async_copy(
                        table_hbm.at[idx_v.at[pl.ds(j * SUB, SUB)]],
                        rows_v.at[pl.ds(j * SUB, SUB)], sem))
            for cp in copies:
                cp.wait()

            def repack(r, c2):
                for u in range(8):
                    packed_v[r, pl.ds(16 * u, 16)] = rows_v[8 * r + u, :]
                return c2

            lax.fori_loop(0, PROW, repack, 0)

            @pl.when(c < LT_CHUNKS)
            def _():
                pltpu.sync_copy(packed_v, lt_hbm.at[pl.ds(c * PROW, PROW)])

            @pl.when(c >= LT_CHUNKS)
            def _():
                pltpu.sync_copy(
                    packed_v,
                    it_hbm.at[pl.ds((c - LT_CHUNKS) * PROW, PROW)])

            return carry

        lax.fori_loop(0, NCHUNK, body, 0)

    return k(table, ids)


BB = 128
NR = BB // Q
B8 = B // Q


def _tc_combine(xlt, xit, maskt, w_all, tg, e8, bd):
    def kern(x_ref, ix_ref, mk_ref, w_ref, tg_ref, e8_ref, bd_ref, out_ref):
        x3 = x_ref[...]
        ix3 = ix_ref[...]
        mkt = mk_ref[0]
        w = w_ref[...]
        tg = tg_ref[...]
        e8 = e8_ref[...]
        for r in range(NR):
            xr = x3[:, r, :]
            ixr = ix3[:, r, :]
            y = jnp.zeros((L, Q * GM), jnp.float32)
            yi = jnp.zeros((S, Q * GM), jnp.float32)
            for f in range(F):
                y = y + lax.dot_general(
                    xr[L * f:L * (f + 1), :], w[f],
                    (((1,), (0,)), ((), ())),
                    preferred_element_type=jnp.float32)
                yi = yi + lax.dot_general(
                    ixr[S * f:S * (f + 1), :], w[f],
                    (((1,), (0,)), ((), ())),
                    preferred_element_type=jnp.float32)
            sb = jnp.where(y > 0, 1.0, -1.0)
            oh = (lax.dot_general(sb, tg, (((1,), (0,)), ((), ())),
                                  preferred_element_type=jnp.float32)
                  > GL - 0.5).astype(jnp.float32)
            sbi = jnp.where(yi > 0, 1.0, -1.0)
            ohi = (lax.dot_general(sbi, tg, (((1,), (0,)), ((), ())),
                                   preferred_element_type=jnp.float32)
                   > GL - 0.5).astype(jnp.float32)
            valid = jnp.sum(oh, axis=0)
            inv = (1.0 / G) / jnp.maximum(valid, 1.0)
            mk_big = lax.dot_general(
                mkt[:, Q * r:Q * (r + 1)], e8, (((1,), (0,)), ((), ())),
                preferred_element_type=jnp.float32)
            ohm = oh * mk_big
            its = ohi * inv[None, :]
            bd = bd_ref[...]
            ofs = []
            for f in range(F):
                xf = xr[L * f:L * (f + 1), :]
                a_full = lax.dot_general(
                    ohm, xf, (((0,), (0,)), ((), ())),
                    preferred_element_type=jnp.float32)
                ofs.append(lax.dot_general(
                    its, a_full * bd, (((1,), (0,)), ((), ())),
                    preferred_element_type=jnp.float32))
            oqs = []
            for q in range(Q):
                oqs.append(jnp.concatenate(
                    [ofs[f][:, EDIM * q:EDIM * (q + 1)] for f in range(F)],
                    axis=1))
            out_ref[r] = jnp.stack(oqs, axis=0)

    return pl.pallas_call(
        kern,
        grid=(B // BB,),
        in_specs=[
            pl.BlockSpec((F * L, NR, 128), lambda i: (0, i, 0)),
            pl.BlockSpec((F * S, NR, 128), lambda i: (0, i, 0)),
            pl.BlockSpec((1, L, BB), lambda i: (i, 0, 0)),
            pl.BlockSpec((F, 128, Q * GM), lambda i: (0, 0, 0)),
            pl.BlockSpec((Q * GM, Q * GC), lambda i: (0, 0)),
            pl.BlockSpec((Q, Q * GC), lambda i: (0, 0)),
            pl.BlockSpec((Q * GC, 128), lambda i: (0, 0)),
        ],
        out_specs=pl.BlockSpec((NR, Q, S, EXT), lambda i: (i, 0, 0, 0)),
        out_shape=jax.ShapeDtypeStruct((B8, Q, S, EXT), jnp.float32),
    )(xlt, xit, maskt, w_all, tg, e8, bd)


def _make_t_sel():
    t = np.zeros((GM, GC), np.float32)
    for g in range(G):
        for c in range(MC):
            for m in range(GL):
                t[GL * g + m, MC * g + c] = 1.0 if (c >> m) & 1 else -1.0
    return t


_TG = np.kron(np.eye(Q, dtype=np.float32), _make_t_sel())
_E8 = np.kron(np.eye(Q, dtype=np.float32),
              np.ones((1, GC), np.float32))
_BD = np.kron(np.eye(Q, dtype=np.float32),
              np.ones((GC, EDIM), np.float32))


def kernel(item_ids, longterm_ids, longterm_mask, embed_table, H):
    lt_t = jnp.transpose(longterm_ids, (2, 1, 0))
    it_t = jnp.transpose(item_ids, (2, 1, 0))
    ids = jnp.concatenate([lt_t.reshape(-1), it_t.reshape(-1)])
    out_lt, out_it = _sc_gather_packed(embed_table, ids.astype(jnp.int32))
    xlt = out_lt.reshape(F * L, B8, 128)
    xit = out_it.reshape(F * S, B8, 128)
    maskt = longterm_mask.T.astype(jnp.float32)
    maskt = maskt.reshape(L, B // BB, BB).transpose(1, 0, 2)
    h2 = H.reshape(EXT, GM)
    w_all = jnp.stack([
        jnp.kron(jnp.eye(Q, dtype=jnp.float32), h2[EDIM * f:EDIM * (f + 1)])
        for f in range(F)])
    out = _tc_combine(xlt, xit, maskt, w_all, jnp.asarray(_TG),
                      jnp.asarray(_E8), jnp.asarray(_BD))
    return out.reshape(B, S, EXT)


# speedup vs baseline: 2.0842x; 1.0771x over previous
import functools

import jax
import jax.numpy as jnp
import numpy as np
from jax import lax
from jax.experimental import pallas as pl
from jax.experimental.pallas import tpu as pltpu
from jax.experimental.pallas import tpu_sc as plsc

B, S, L, F = 1024, 8, 200, 3
EDIM = 16
EXT = F * EDIM
G, GL = 4, 3
GM = G * GL
MC = 2 ** GL
GC = G * MC
Q = 8

N_LT = B * L * F
N_IT = B * S * F
N_ROWS = N_LT + N_IT

NW = 32
ROWS_PER_W = N_ROWS // NW
SUB = 128
NSUB = 12
CHUNK = SUB * NSUB
NCHUNK = ROWS_PER_W // CHUNK
PROW = CHUNK * EDIM // 128
LT_CHUNKS = N_LT // CHUNK
P_LT = N_LT * EDIM // 128
P_IT = N_IT * EDIM // 128


def _sc_gather_packed(table, ids_lt, ids_it):
    mesh = plsc.VectorSubcoreMesh(core_axis_name="c", subcore_axis_name="s")

    @functools.partial(
        pl.kernel,
        out_type=[jax.ShapeDtypeStruct((P_LT, 128), jnp.float32),
                  jax.ShapeDtypeStruct((P_IT, 128), jnp.float32)],
        mesh=mesh,
        scratch_types=[
            pltpu.VMEM((2, CHUNK), jnp.int32),
            pltpu.VMEM((2, CHUNK, EDIM), jnp.float32),
            pltpu.VMEM((PROW, 128), jnp.float32),
            pltpu.SemaphoreType.DMA,
            pltpu.SemaphoreType.DMA,
        ],
        compiler_params=pltpu.CompilerParams(use_tc_tiling_on_sc=False),
    )
    def k(table_hbm, lt_ids_hbm, it_ids_hbm, lt_hbm, it_hbm,
          idx_v, rows_v, packed_v, sem0, sem1):
        wid = lax.axis_index("s") * 2 + lax.axis_index("c")
        sems = [sem0, sem1]

        def stage(i, buf):
            c = wid * NCHUNK + i

            @pl.when(c < LT_CHUNKS)
            def _():
                pltpu.sync_copy(lt_ids_hbm.at[pl.ds(c * CHUNK, CHUNK)],
                                idx_v.at[buf])

            @pl.when(c >= LT_CHUNKS)
            def _():
                pltpu.sync_copy(
                    it_ids_hbm.at[pl.ds((c - LT_CHUNKS) * CHUNK, CHUNK)],
                    idx_v.at[buf])

            return [pltpu.async_copy(
                table_hbm.at[idx_v.at[buf, pl.ds(j * SUB, SUB)]],
                rows_v.at[buf, pl.ds(j * SUB, SUB)], sems[buf])
                for j in range(NSUB)]

        cur = stage(0, 0)
        for i in range(NCHUNK):
            buf = i % 2
            nxt = None
            if i + 1 < NCHUNK:
                nxt = stage(i + 1, 1 - buf)
            for cp in cur:
                cp.wait()

            def repack(r, c2, buf=buf):
                for u in range(8):
                    packed_v[r, pl.ds(16 * u, 16)] = rows_v[buf, 8 * r + u, :]
                return c2

            lax.fori_loop(0, PROW, repack, 0)
            c = wid * NCHUNK + i

            @pl.when(c < LT_CHUNKS)
            def _():
                pltpu.sync_copy(packed_v, lt_hbm.at[pl.ds(c * PROW, PROW)])

            @pl.when(c >= LT_CHUNKS)
            def _():
                pltpu.sync_copy(
                    packed_v,
                    it_hbm.at[pl.ds((c - LT_CHUNKS) * PROW, PROW)])

            cur = nxt

    return k(table, ids_lt, ids_it)


BB = 128
NR = BB // Q
B8 = B // Q


def _tc_combine(xlt, xit, maskt, w_all, tg, e8, bd):
    def kern(x_ref, ix_ref, mk_ref, w_ref, tg_ref, e8_ref, bd_ref, out_ref):
        x3 = x_ref[...]
        ix3 = ix_ref[...]
        mkt = mk_ref[0]
        w = w_ref[...]
        tg = tg_ref[...]
        e8 = e8_ref[...]
        for r in range(NR):
            xr = x3[:, r, :]
            ixr = ix3[:, r, :]
            y = jnp.zeros((L, Q * GM), jnp.float32)
            yi = jnp.zeros((S, Q * GM), jnp.float32)
            for f in range(F):
                y = y + lax.dot_general(
                    xr[L * f:L * (f + 1), :], w[f],
                    (((1,), (0,)), ((), ())),
                    preferred_element_type=jnp.float32)
                yi = yi + lax.dot_general(
                    ixr[S * f:S * (f + 1), :], w[f],
                    (((1,), (0,)), ((), ())),
                    preferred_element_type=jnp.float32)
            sb = jnp.where(y > 0, 1.0, -1.0)
            oh = (lax.dot_general(sb, tg, (((1,), (0,)), ((), ())),
                                  preferred_element_type=jnp.float32)
                  > GL - 0.5).astype(jnp.float32)
            sbi = jnp.where(yi > 0, 1.0, -1.0)
            ohi = (lax.dot_general(sbi, tg, (((1,), (0,)), ((), ())),
                                   preferred_element_type=jnp.float32)
                   > GL - 0.5).astype(jnp.float32)
            valid = jnp.sum(oh, axis=0)
            inv = (1.0 / G) / jnp.maximum(valid, 1.0)
            mk_big = lax.dot_general(
                mkt[:, Q * r:Q * (r + 1)], e8, (((1,), (0,)), ((), ())),
                preferred_element_type=jnp.float32)
            ohm = oh * mk_big
            its = ohi * inv[None, :]
            bd = bd_ref[...]
            ofs = []
            for f in range(F):
                xf = xr[L * f:L * (f + 1), :]
                a_full = lax.dot_general(
                    ohm, xf, (((0,), (0,)), ((), ())),
                    preferred_element_type=jnp.float32)
                ofs.append(lax.dot_general(
                    its, a_full * bd, (((1,), (0,)), ((), ())),
                    preferred_element_type=jnp.float32))
            oqs = []
            for q in range(Q):
                oqs.append(jnp.concatenate(
                    [ofs[f][:, EDIM * q:EDIM * (q + 1)] for f in range(F)],
                    axis=1))
            out_ref[r] = jnp.stack(oqs, axis=0)

    return pl.pallas_call(
        kern,
        grid=(B // BB,),
        in_specs=[
            pl.BlockSpec((F * L, NR, 128), lambda i: (0, i, 0)),
            pl.BlockSpec((F * S, NR, 128), lambda i: (0, i, 0)),
            pl.BlockSpec((1, L, BB), lambda i: (i, 0, 0)),
            pl.BlockSpec((F, 128, Q * GM), lambda i: (0, 0, 0)),
            pl.BlockSpec((Q * GM, Q * GC), lambda i: (0, 0)),
            pl.BlockSpec((Q, Q * GC), lambda i: (0, 0)),
            pl.BlockSpec((Q * GC, 128), lambda i: (0, 0)),
        ],
        out_specs=pl.BlockSpec((NR, Q, S, EXT), lambda i: (i, 0, 0, 0)),
        out_shape=jax.ShapeDtypeStruct((B8, Q, S, EXT), jnp.float32),
    )(xlt, xit, maskt, w_all, tg, e8, bd)


def _make_t_sel():
    t = np.zeros((GM, GC), np.float32)
    for g in range(G):
        for c in range(MC):
            for m in range(GL):
                t[GL * g + m, MC * g + c] = 1.0 if (c >> m) & 1 else -1.0
    return t


_TG = np.kron(np.eye(Q, dtype=np.float32), _make_t_sel())
_E8 = np.kron(np.eye(Q, dtype=np.float32),
              np.ones((1, GC), np.float32))
_BD = np.kron(np.eye(Q, dtype=np.float32),
              np.ones((GC, EDIM), np.float32))


def kernel(item_ids, longterm_ids, longterm_mask, embed_table, H):
    lt_t = jnp.transpose(longterm_ids, (2, 1, 0))
    it_t = jnp.transpose(item_ids, (2, 1, 0))
    out_lt, out_it = _sc_gather_packed(
        embed_table, lt_t.reshape(-1), it_t.reshape(-1))
    xlt = out_lt.reshape(F * L, B8, 128)
    xit = out_it.reshape(F * S, B8, 128)
    maskt = longterm_mask.T.astype(jnp.float32)
    maskt = maskt.reshape(L, B // BB, BB).transpose(1, 0, 2)
    h2 = H.reshape(EXT, GM)
    w_all = jnp.stack([
        jnp.kron(jnp.eye(Q, dtype=jnp.float32), h2[EDIM * f:EDIM * (f + 1)])
        for f in range(F)])
    out = _tc_combine(xlt, xit, maskt, w_all, jnp.asarray(_TG),
                      jnp.asarray(_E8), jnp.asarray(_BD))
    return out.reshape(B, S, EXT)
